# Initial kernel scaffold; baseline (speedup 1.0000x reference)
import jax
import jax.numpy as jnp
from jax.experimental import pallas as pl


def kernel(seq_len, pe):
    raise NotImplementedError("write your pallas kernel here")


# speedup vs baseline: 1.6947x; 1.6947x over previous
import functools

import jax
import jax.numpy as jnp
from jax import lax
from jax.experimental import pallas as pl
from jax.experimental.pallas import tpu as pltpu
from jax.experimental.pallas import tpu_sc as plsc

_NUM_CORES = 2
_NUM_SUBCORES = 16
_NW = _NUM_CORES * _NUM_SUBCORES
_CHUNK = 128


@functools.cache
def _make_gather(n_rows: int, d: int):
    rows_per_w = -(-n_rows // _NW)
    rows_per_w = -(-rows_per_w // _CHUNK) * _CHUNK
    n_pad = rows_per_w * _NW
    chunks = rows_per_w // _CHUNK
    tail = n_rows - (_NW - 1) * rows_per_w

    mesh = plsc.VectorSubcoreMesh(core_axis_name="c", subcore_axis_name="s")

    @functools.partial(
        pl.kernel,
        mesh=mesh,
        out_type=jax.ShapeDtypeStruct((n_rows, d), jnp.float32),
        compiler_params=pltpu.CompilerParams(use_tc_tiling_on_sc=False),
        scratch_types=[
            pltpu.VMEM((chunks, _CHUNK), jnp.int32),
            pltpu.VMEM((rows_per_w, d), jnp.float32),
            pltpu.SemaphoreType.DMA,
        ],
    )
    def gather_kernel(pe_hbm, idx_hbm, out_hbm, idx_v, rows_v, sem):
        wid = lax.axis_index("s") * _NUM_CORES + lax.axis_index("c")
        base = wid * rows_per_w
        pltpu.sync_copy(idx_hbm.at[pl.ds(wid * chunks, chunks)], idx_v)
        copies = []
        for c in range(chunks):
            copies.append(
                pltpu.async_copy(
                    pe_hbm.at[idx_v.at[c]],
                    rows_v.at[pl.ds(c * _CHUNK, _CHUNK)],
                    sem,
                )
            )
        for cp in copies:
            cp.wait()

        @pl.when(wid < _NW - 1)
        def _():
            pltpu.sync_copy(rows_v, out_hbm.at[pl.ds(base, rows_per_w)])

        @pl.when(wid == _NW - 1)
        def _():
            pltpu.sync_copy(
                rows_v.at[pl.ds(0, tail)], out_hbm.at[pl.ds(base, tail)]
            )

    return gather_kernel, n_pad


def kernel(seq_len, pe):
    n, d = pe.shape
    static_len = (n + 1) // 2
    gather, n_pad = _make_gather(n, d)
    offset = jnp.asarray(seq_len, jnp.int32) - static_len
    idx = jnp.clip(
        jnp.arange(n_pad, dtype=jnp.int32) + offset, 0, n - 1
    ).reshape(n_pad // _CHUNK, _CHUNK)
    return gather(pe, idx)
